# baseline (speedup 1.0000x reference)
import functools

import jax
import jax.numpy as jnp
from jax import lax
from jax.experimental import pallas as pl
from jax.experimental.pallas import tpu as pltpu
from jax.experimental.pallas import tpu_sc as plsc

_N_STATES = 100000
_ALPHABET = 128
_BATCH = 16384

_NC = 2
_NS = 16
_NW = _NC * _NS
_B_PER_W = _BATCH // _NW
_CHUNK = 128
_NCHUNK = _B_PER_W // _CHUNK
_LANES = 16


def _predictor_kernel(inp_hbm, hid_hbm, trans_hbm, pred_hbm,
                      res_hbm, nh_hbm,
                      hid_v, inp_v, idx_v, nh_v, rows_v,
                      sem_i, sem_r, sem_o):
    wid = lax.axis_index("s") * _NC + lax.axis_index("c")
    base = wid * _B_PER_W

    pltpu.sync_copy(hid_hbm.at[pl.ds(base, _B_PER_W)], hid_v)
    pltpu.sync_copy(inp_hbm.at[pl.ds(base, _B_PER_W)], inp_v)

    for i in range(_B_PER_W // _LANES):
        h = hid_v[pl.ds(i * _LANES, _LANES)]
        c = inp_v[pl.ds(i * _LANES, _LANES)]
        row = i // (_CHUNK // _LANES)
        col = (i % (_CHUNK // _LANES)) * _LANES
        idx_v[row, pl.ds(col, _LANES)] = h * _ALPHABET + c

    nh_copies = [
        pltpu.async_copy(trans_hbm.at[idx_v.at[j]], nh_v.at[j], sem_i)
        for j in range(_NCHUNK)
    ]
    row_copies = []
    for j in range(_NCHUNK):
        nh_copies[j].wait()
        row_copies.append(
            pltpu.async_copy(pred_hbm.at[nh_v.at[j]],
                             rows_v.at[pl.ds(j * _CHUNK, _CHUNK)], sem_r))

    out_copies = [pltpu.async_copy(nh_v, nh_hbm.at[wid], sem_o)]
    for j in range(_NCHUNK):
        row_copies[j].wait()
        out_copies.append(
            pltpu.async_copy(rows_v.at[pl.ds(j * _CHUNK, _CHUNK)],
                             res_hbm.at[pl.ds(base + j * _CHUNK, _CHUNK)],
                             sem_o))
    for c in out_copies:
        c.wait()


@jax.jit
def _run(inp, hid, trans_flat, pred):
    mesh = plsc.VectorSubcoreMesh(core_axis_name="c", subcore_axis_name="s")
    f = pl.kernel(
        _predictor_kernel,
        mesh=mesh,
        out_type=(
            jax.ShapeDtypeStruct((_BATCH, _ALPHABET), jnp.float32),
            jax.ShapeDtypeStruct((_NW, _NCHUNK, _CHUNK), jnp.int32),
        ),
        scratch_types=[
            pltpu.VMEM((_B_PER_W,), jnp.int32),
            pltpu.VMEM((_B_PER_W,), jnp.int32),
            pltpu.VMEM((_NCHUNK, _CHUNK), jnp.int32),
            pltpu.VMEM((_NCHUNK, _CHUNK), jnp.int32),
            pltpu.VMEM((_B_PER_W, _ALPHABET), jnp.float32),
            pltpu.SemaphoreType.DMA,
            pltpu.SemaphoreType.DMA,
            pltpu.SemaphoreType.DMA,
        ],
    )
    return f(inp, hid, trans_flat, pred)


def kernel(input, hidden, transitions, predictions):
    trans_flat = transitions.reshape(-1)
    result, nh = _run(input, hidden, trans_flat, predictions)
    return result, nh.reshape(-1)


# speedup vs baseline: 1.0198x; 1.0198x over previous
import functools

import jax
import jax.numpy as jnp
from jax import lax
from jax.experimental import pallas as pl
from jax.experimental.pallas import tpu as pltpu
from jax.experimental.pallas import tpu_sc as plsc

_N_STATES = 100000
_ALPHABET = 128
_BATCH = 16384

_NC = 2
_NS = 16
_NW = _NC * _NS
_B_PER_W = _BATCH // _NW
_CHUNK = 128
_NCHUNK = _B_PER_W // _CHUNK
_LANES = 16


def _predictor_kernel(inp_hbm, hid_hbm, trans_hbm, pred_hbm,
                      res_hbm, nh_hbm,
                      hid_v, inp_v, idx_v, nh_v, rows_v,
                      sem_i, sem_r, sem_o):
    wid = lax.axis_index("s") * _NC + lax.axis_index("c")
    base = wid * _B_PER_W

    ld_h = pltpu.async_copy(hid_hbm.at[pl.ds(base, _B_PER_W)], hid_v, sem_i)
    ld_c = pltpu.async_copy(inp_hbm.at[pl.ds(base, _B_PER_W)], inp_v, sem_i)
    ld_h.wait()
    ld_c.wait()

    def _flat_idx(i, carry):
        j = i // (_CHUNK // _LANES)
        col = (i % (_CHUNK // _LANES)) * _LANES
        h = hid_v[pl.ds(i * _LANES, _LANES)]
        c = inp_v[pl.ds(i * _LANES, _LANES)]
        idx_v[j, pl.ds(col, _LANES)] = h * _ALPHABET + c
        return carry

    lax.fori_loop(0, _B_PER_W // _LANES, _flat_idx, 0)

    nh_copies = [
        pltpu.async_copy(trans_hbm.at[idx_v.at[j]], nh_v.at[j], sem_i)
        for j in range(_NCHUNK)
    ]
    row_copies = []
    for j in range(_NCHUNK):
        nh_copies[j].wait()
        row_copies.append(
            pltpu.async_copy(pred_hbm.at[nh_v.at[j]],
                             rows_v.at[pl.ds(j * _CHUNK, _CHUNK)], sem_r))

    out_copies = [pltpu.async_copy(nh_v, nh_hbm.at[wid], sem_o)]
    for j in range(_NCHUNK):
        row_copies[j].wait()
        out_copies.append(
            pltpu.async_copy(rows_v.at[pl.ds(j * _CHUNK, _CHUNK)],
                             res_hbm.at[pl.ds(base + j * _CHUNK, _CHUNK)],
                             sem_o))
    for c in out_copies:
        c.wait()


@jax.jit
def _run(inp, hid, trans_flat, pred):
    mesh = plsc.VectorSubcoreMesh(core_axis_name="c", subcore_axis_name="s")
    f = pl.kernel(
        _predictor_kernel,
        mesh=mesh,
        out_type=(
            jax.ShapeDtypeStruct((_BATCH, _ALPHABET), jnp.float32),
            jax.ShapeDtypeStruct((_NW, _NCHUNK, _CHUNK), jnp.int32),
        ),
        scratch_types=[
            pltpu.VMEM((_B_PER_W,), jnp.int32),
            pltpu.VMEM((_B_PER_W,), jnp.int32),
            pltpu.VMEM((_NCHUNK, _CHUNK), jnp.int32),
            pltpu.VMEM((_NCHUNK, _CHUNK), jnp.int32),
            pltpu.VMEM((_B_PER_W, _ALPHABET), jnp.float32),
            pltpu.SemaphoreType.DMA,
            pltpu.SemaphoreType.DMA,
            pltpu.SemaphoreType.DMA,
        ],
    )
    return f(inp, hid, trans_flat, pred)


def kernel(input, hidden, transitions, predictions):
    trans_flat = transitions.reshape(-1)
    result, nh = _run(input, hidden, trans_flat, predictions)
    return result, nh.reshape(-1)
